# baseline (speedup 1.0000x reference)
import functools

import jax
import jax.numpy as jnp
from jax import lax
from jax.experimental import pallas as pl
from jax.experimental.pallas import tpu as pltpu
from jax.experimental.pallas import tpu_sc as plsc

BATCH = 16384
EMBED_DIM = 64
FEAT_DIM = 64
DENSE_DIM = 32

NUM_CORES = 2
NUM_SUBCORES = 16
NUM_WORKERS = NUM_CORES * NUM_SUBCORES
B_PER_W = BATCH // NUM_WORKERS
GATHER_CHUNK = 128
N_CHUNKS = B_PER_W // GATHER_CHUNK


def _sc_gather_pair(user_table, user_id, item_table, video_id):
    mesh = plsc.VectorSubcoreMesh(core_axis_name="c", subcore_axis_name="s")
    out_t = (
        jax.ShapeDtypeStruct((BATCH, EMBED_DIM), jnp.float32),
        jax.ShapeDtypeStruct((BATCH, EMBED_DIM), jnp.float32),
    )

    @functools.partial(
        pl.kernel,
        out_type=out_t,
        mesh=mesh,
        compiler_params=pltpu.CompilerParams(use_tc_tiling_on_sc=False),
        scratch_types=[
            pltpu.VMEM((B_PER_W,), jnp.int32),
            pltpu.VMEM((B_PER_W,), jnp.int32),
            pltpu.VMEM((B_PER_W, EMBED_DIM), jnp.float32),
            pltpu.VMEM((B_PER_W, EMBED_DIM), jnp.float32),
            pltpu.SemaphoreType.DMA,
            pltpu.SemaphoreType.DMA,
        ],
    )
    def k(ut_hbm, uid_hbm, it_hbm, vid_hbm, uout_hbm, iout_hbm,
          idx_u, idx_i, rows_u, rows_i, sem_u, sem_i):
        wid = lax.axis_index("s") * NUM_CORES + lax.axis_index("c")
        base = wid * B_PER_W
        pltpu.sync_copy(uid_hbm.at[pl.ds(base, B_PER_W)], idx_u)
        pltpu.sync_copy(vid_hbm.at[pl.ds(base, B_PER_W)], idx_i)
        for c in range(N_CHUNKS):
            sl = pl.ds(c * GATHER_CHUNK, GATHER_CHUNK)
            pltpu.async_copy(ut_hbm.at[idx_u.at[sl]], rows_u.at[sl], sem_u)
            pltpu.async_copy(it_hbm.at[idx_i.at[sl]], rows_i.at[sl], sem_i)
        for c in range(N_CHUNKS):
            sl = pl.ds(c * GATHER_CHUNK, GATHER_CHUNK)
            pltpu.make_async_copy(ut_hbm.at[idx_u.at[sl]], rows_u.at[sl],
                                  sem_u).wait()
            pltpu.make_async_copy(it_hbm.at[idx_i.at[sl]], rows_i.at[sl],
                                  sem_i).wait()
        pltpu.sync_copy(rows_u, uout_hbm.at[pl.ds(base, B_PER_W)])
        pltpu.sync_copy(rows_i, iout_hbm.at[pl.ds(base, B_PER_W)])

    return k(user_table, user_id, item_table, video_id)


BLK = 2048


def _tc_body(ue_ref, ie_ref, uf_ref, vf_ref, wu_ref, bu_ref, wi_ref, bi_ref,
             out_ref):
    u_feat = jnp.maximum(
        jnp.dot(uf_ref[...], wu_ref[...],
                preferred_element_type=jnp.float32) + bu_ref[...], 0.0)
    i_feat = jnp.maximum(
        jnp.dot(vf_ref[...], wi_ref[...],
                preferred_element_type=jnp.float32) + bi_ref[...], 0.0)
    dot = (jnp.sum(ue_ref[...] * ie_ref[...], axis=1)
           + jnp.sum(u_feat * i_feat, axis=1))
    out_ref[...] = dot[None, :]


def _tc_combine(u_emb, i_emb, user_features, video_features, Wu, bu, Wi, bi):
    grid = (BATCH // BLK,)
    bspec_b = pl.BlockSpec((BLK, FEAT_DIM), lambda i: (i, 0))
    bspec_w = pl.BlockSpec((FEAT_DIM, DENSE_DIM), lambda i: (0, 0))
    bspec_bias = pl.BlockSpec((1, DENSE_DIM), lambda i: (0, 0))
    out = pl.pallas_call(
        _tc_body,
        grid=grid,
        in_specs=[bspec_b, bspec_b, bspec_b, bspec_b,
                  bspec_w, bspec_bias, bspec_w, bspec_bias],
        out_specs=pl.BlockSpec((1, BLK), lambda i: (0, i)),
        out_shape=jax.ShapeDtypeStruct((1, BATCH), jnp.float32),
    )(u_emb, i_emb, user_features, video_features,
      Wu, bu.reshape(1, DENSE_DIM), Wi, bi.reshape(1, DENSE_DIM))
    return out.reshape(BATCH)


@jax.jit
def kernel(user_id, user_features, video_id, video_features, user_table,
           item_table, Wu, bu, Wi, bi):
    u_emb, i_emb = _sc_gather_pair(user_table, user_id.astype(jnp.int32),
                                   item_table, video_id.astype(jnp.int32))
    return _tc_combine(u_emb, i_emb, user_features, video_features,
                       Wu, bu, Wi, bi)
